# Initial kernel scaffold; baseline (speedup 1.0000x reference)
import jax
import jax.numpy as jnp
from jax.experimental import pallas as pl


def kernel(x, edge_index, batch, sex, cag, W0, a_src0, a_dst0, b0, W1, a_src1, a_dst1, b1, lin1_W, lin1_b, lin2_W, lin2_b):
    raise NotImplementedError("write your pallas kernel here")


# speedup vs baseline: 16.4750x; 16.4750x over previous
import functools

import jax
import jax.numpy as jnp
from jax import lax
from jax.experimental import pallas as pl
from jax.experimental.pallas import tpu as pltpu
from jax.experimental.pallas import tpu_sc as plsc

N_ = 10000
E_ = 320000
H_ = 128
HH = 64
NC = 2
NS = 16
L = 16
K = 80
EPC = E_ // NS
NCHUNK = EPC // K
RPT = N_ // NS
ZR = 125
BM = 1000


def _mm2_body(x_ref, w_ref, wa_ref, ha_ref, hb_ref, aux_ref):
    xb = x_ref[...]
    h = jnp.dot(xb, w_ref[...], preferred_element_type=jnp.float32)
    ha_ref[...] = h[:, :HH]
    hb_ref[...] = h[:, HH:]
    aux_ref[...] = jnp.dot(xb, wa_ref[...], preferred_element_type=jnp.float32)


def _mm2(x, W, Waux):
    n = x.shape[0]
    return pl.pallas_call(
        _mm2_body,
        grid=(n // BM,),
        in_specs=[
            pl.BlockSpec((BM, H_), lambda i: (i, 0)),
            pl.BlockSpec((H_, H_), lambda i: (0, 0)),
            pl.BlockSpec((H_, H_), lambda i: (0, 0)),
        ],
        out_specs=[
            pl.BlockSpec((BM, HH), lambda i: (i, 0)),
            pl.BlockSpec((BM, HH), lambda i: (i, 0)),
            pl.BlockSpec((BM, H_), lambda i: (i, 0)),
        ],
        out_shape=[
            jax.ShapeDtypeStruct((n, HH), jnp.float32),
            jax.ShapeDtypeStruct((n, HH), jnp.float32),
            jax.ShapeDtypeStruct((n, H_), jnp.float32),
        ],
    )(x, W, Waux)


def _mid_body(na, nb, dt, b_ref, w_ref, wa_ref, ha_ref, hb_ref, aux_ref):
    d0 = jnp.sum(dt[...], axis=1, keepdims=True) + 1e-16
    h = jnp.concatenate([na[...], nb[...]], axis=1) / d0 + b_ref[...]
    h = jnp.where(h > 0, h, jnp.exp(h) - 1.0)
    h1 = jnp.dot(h, w_ref[...], preferred_element_type=jnp.float32)
    ha_ref[...] = h1[:, :HH]
    hb_ref[...] = h1[:, HH:]
    aux_ref[...] = jnp.dot(h, wa_ref[...], preferred_element_type=jnp.float32)


def _mid(num, dent, b, W, Waux):
    return pl.pallas_call(
        _mid_body,
        grid=(N_ // BM,),
        in_specs=[
            pl.BlockSpec((BM, HH), lambda i: (i, 0)),
            pl.BlockSpec((BM, HH), lambda i: (i, 0)),
            pl.BlockSpec((BM, NS), lambda i: (i, 0)),
            pl.BlockSpec((1, H_), lambda i: (0, 0)),
            pl.BlockSpec((H_, H_), lambda i: (0, 0)),
            pl.BlockSpec((H_, H_), lambda i: (0, 0)),
        ],
        out_specs=[
            pl.BlockSpec((BM, HH), lambda i: (i, 0)),
            pl.BlockSpec((BM, HH), lambda i: (i, 0)),
            pl.BlockSpec((BM, H_), lambda i: (i, 0)),
        ],
        out_shape=[
            jax.ShapeDtypeStruct((N_, HH), jnp.float32),
            jax.ShapeDtypeStruct((N_, HH), jnp.float32),
            jax.ShapeDtypeStruct((N_, H_), jnp.float32),
        ],
    )(num[0], num[1], dent, b, W, Waux)


def _head_body(na, nb, dt, b_ref, m_ref, w1_ref, b1_ref, w2_ref, wex_ref,
               ex_ref, b2_ref, pred_ref, xl_ref, sums, counts):
    i = pl.program_id(0)

    @pl.when(i == 0)
    def _():
        sums[...] = jnp.zeros_like(sums)
        counts[...] = jnp.zeros_like(counts)

    d0 = jnp.sum(dt[...], axis=1, keepdims=True) + 1e-16
    h = jnp.concatenate([na[...], nb[...]], axis=1) / d0 + b_ref[...]
    m = m_ref[...].reshape(m_ref.shape[1], m_ref.shape[2])
    sums[...] += jnp.dot(m, h, preferred_element_type=jnp.float32)
    counts[...] += jnp.broadcast_to(jnp.sum(m, axis=1, keepdims=True), counts.shape)

    @pl.when(i == pl.num_programs(0) - 1)
    def _():
        pooled = sums[...] / jnp.maximum(counts[...], 1.0)
        xl = jnp.dot(pooled, w1_ref[...], preferred_element_type=jnp.float32)
        xl = jnp.maximum(xl + b1_ref[...], 0.0)
        xl_ref[...] = xl
        logits = (jnp.dot(xl, w2_ref[...], preferred_element_type=jnp.float32)
                  + jnp.dot(ex_ref[...], wex_ref[...], preferred_element_type=jnp.float32)
                  + b2_ref[...])
        ci = lax.broadcasted_iota(jnp.int32, logits.shape, 1)
        logits = jnp.where(ci < 5, logits, -1e30)
        mx = jnp.max(logits, axis=1, keepdims=True)
        p = jnp.exp(logits - mx)
        pred_ref[...] = p / jnp.sum(p, axis=1, keepdims=True)


def _head(num, dent, b, msk, w1, b1, w2, wex, exin, b2):
    g = msk.shape[1]
    return pl.pallas_call(
        _head_body,
        grid=(N_ // BM,),
        in_specs=[
            pl.BlockSpec((BM, HH), lambda i: (i, 0)),
            pl.BlockSpec((BM, HH), lambda i: (i, 0)),
            pl.BlockSpec((BM, NS), lambda i: (i, 0)),
            pl.BlockSpec((1, H_), lambda i: (0, 0)),
            pl.BlockSpec((1, g, BM), lambda i: (i, 0, 0)),
            pl.BlockSpec((H_, H_), lambda i: (0, 0)),
            pl.BlockSpec((1, H_), lambda i: (0, 0)),
            pl.BlockSpec((H_, H_), lambda i: (0, 0)),
            pl.BlockSpec((H_, H_), lambda i: (0, 0)),
            pl.BlockSpec((g, H_), lambda i: (0, 0)),
            pl.BlockSpec((1, H_), lambda i: (0, 0)),
        ],
        out_specs=[
            pl.BlockSpec((g, H_), lambda i: (0, 0)),
            pl.BlockSpec((g, H_), lambda i: (0, 0)),
        ],
        out_shape=[jax.ShapeDtypeStruct((g, H_), jnp.float32)] * 2,
        scratch_shapes=[
            pltpu.VMEM((g, H_), jnp.float32),
            pltpu.VMEM((g, H_), jnp.float32),
        ],
    )(num[0], num[1], dent, b, msk, w1, b1, w2, wex, exin, b2)


def _sc_edge_body(ei_hbm, ha_hbm, hb_hbm, as_hbm, ad_hbm, zr_hbm, num_hbm, den_hbm,
                  as_v, ad_v, src_v, dst_v, rows_v, zbuf, den_v, acc, sem):
    c = lax.axis_index("c")
    s = lax.axis_index("s")

    pltpu.sync_copy(as_hbm, as_v)
    pltpu.sync_copy(ad_hbm, ad_v)
    pltpu.sync_copy(zr_hbm, zbuf)

    def zden(i, cr):
        den_v[pl.ds(i * L, L)] = jnp.zeros((L,), jnp.float32)
        return cr
    lax.fori_loop(0, N_ // L, zden, 0)

    for j in range(RPT // ZR):
        pltpu.sync_copy(zbuf, acc.at[pl.ds(s * RPT + j * ZR, ZR), :])
    plsc.subcore_barrier()

    def chunk(i, carry):
        base = s * EPC + i * K
        pltpu.sync_copy(ei_hbm.at[0, pl.ds(base, K)], src_v)
        pltpu.sync_copy(ei_hbm.at[1, pl.ds(base, K)], dst_v)

        @pl.when(c == 0)
        def _():
            pltpu.async_copy(ha_hbm.at[src_v], rows_v, sem).wait()

        @pl.when(c == 1)
        def _():
            pltpu.async_copy(hb_hbm.at[src_v], rows_v, sem).wait()

        for g in range(K // L):
            si = src_v[pl.ds(g * L, L)]
            di = dst_v[pl.ds(g * L, L)]
            e = plsc.load_gather(as_v, [si]) + plsc.load_gather(ad_v, [di])
            e = jnp.where(e >= 0, e, e * 0.2)
            wvec = jnp.exp(e)
            plsc.addupdate_scatter(den_v, [di], wvec)
            for lane in range(L):
                r = g * L + lane
                wr = wvec[lane]
                for cc in range(HH // L):
                    rows_v[r, pl.ds(cc * L, L)] = rows_v[r, pl.ds(cc * L, L)] * wr
        pltpu.sync_copy(rows_v, acc.at[dst_v], add=True)
        return carry

    lax.fori_loop(0, NCHUNK, chunk, 0)
    plsc.subcore_barrier()

    pltpu.sync_copy(acc.at[pl.ds(s * RPT, RPT), :],
                    num_hbm.at[c, pl.ds(s * RPT, RPT), :])

    @pl.when(c == 0)
    def _():
        pltpu.sync_copy(den_v, den_hbm.at[s])


@functools.cache
def _sc_edge():
    return pl.kernel(
        _sc_edge_body,
        out_type=[
            jax.ShapeDtypeStruct((NC, N_, HH), jnp.float32),
            jax.ShapeDtypeStruct((NS, N_), jnp.float32),
        ],
        mesh=plsc.VectorSubcoreMesh(core_axis_name="c", subcore_axis_name="s",
                                    num_cores=NC, num_subcores=NS),
        compiler_params=pltpu.CompilerParams(use_tc_tiling_on_sc=False,
                                             needs_layout_passes=False),
        scratch_types=[
            pltpu.VMEM((N_,), jnp.float32),
            pltpu.VMEM((N_,), jnp.float32),
            pltpu.VMEM((K,), jnp.int32),
            pltpu.VMEM((K,), jnp.int32),
            pltpu.VMEM((K, HH), jnp.float32),
            pltpu.VMEM((ZR, HH), jnp.float32),
            pltpu.VMEM((N_,), jnp.float32),
            pltpu.VMEM_SHARED((N_, HH), jnp.float32),
            pltpu.SemaphoreType.DMA,
        ],
    )


def _gat_layer_sc(edge_index, ha, hb, a_s, a_d, zr):
    num, den = _sc_edge()(edge_index, ha, hb, a_s, a_d, zr)
    return num, den.T


def kernel(x, edge_index, batch, sex, cag, W0, a_src0, a_dst0, b0,
           W1, a_src1, a_dst1, b1, lin1_W, lin1_b, lin2_W, lin2_b):
    n = x.shape[0]
    g = sex.shape[0]
    f32 = jnp.float32

    wz = jnp.zeros((H_, H_ - 2), f32)
    waux0 = jnp.concatenate([(W0 @ a_src0)[:, None], (W0 @ a_dst0)[:, None], wz], axis=1)
    waux1 = jnp.concatenate([(W1 @ a_src1)[:, None], (W1 @ a_dst1)[:, None], wz], axis=1)
    zr = jnp.zeros((ZR, HH), f32)

    h0a, h0b, aux0 = _mm2(x, W0, waux0)
    num0, dent0 = _gat_layer_sc(edge_index, h0a, h0b, aux0[:, 0], aux0[:, 1], zr)

    h1a, h1b, aux1 = _mid(num0, dent0, b0.reshape(1, H_), W1, waux1)
    num1, dent1 = _gat_layer_sc(edge_index, h1a, h1b, aux1[:, 0], aux1[:, 1], zr)

    msk = (jnp.arange(g, dtype=jnp.int32)[None, :, None]
           == batch.reshape(n // BM, 1, BM)).astype(f32)
    exin = jnp.concatenate([sex[:, None], cag[:, None], jnp.zeros((g, H_ - 2), f32)],
                           axis=1)
    lin2m = jnp.pad(lin2_W[:H_], ((0, 0), (0, H_ - 5)))
    wex = jnp.pad(lin2_W[H_:H_ + 2], ((0, H_ - 2), (0, H_ - 5)))
    lin2b_pad = jnp.pad(lin2_b, (0, H_ - 5))[None, :]
    pred_pad, x_lin1 = _head(num1, dent1, b1.reshape(1, H_), msk,
                             lin1_W, lin1_b.reshape(1, H_), lin2m, wex, exin,
                             lin2b_pad)
    prediction = pred_pad[:, :5]
    return (prediction, x_lin1)


# speedup vs baseline: 38.8519x; 2.3582x over previous
import functools

import jax
import jax.numpy as jnp
from jax import lax
from jax.experimental import pallas as pl
from jax.experimental.pallas import tpu as pltpu
from jax.experimental.pallas import tpu_sc as plsc

N_ = 10000
E_ = 320000
H_ = 128
HH = 64
NC = 2
NS = 16
L = 16
K = 80
EPC = E_ // NS
NCHUNK = EPC // K
RPT = N_ // NS
ZR = 125
BM = 1000


def _mm2_body(x_ref, w_ref, wa_ref, ha_ref, hb_ref, aux_ref):
    xb = x_ref[...]
    h = jnp.dot(xb, w_ref[...], preferred_element_type=jnp.float32)
    ha_ref[...] = h[:, :HH]
    hb_ref[...] = h[:, HH:]
    aux_ref[...] = jnp.dot(xb, wa_ref[...], preferred_element_type=jnp.float32)


def _mm2(x, W, Waux):
    n = x.shape[0]
    return pl.pallas_call(
        _mm2_body,
        grid=(n // BM,),
        in_specs=[
            pl.BlockSpec((BM, H_), lambda i: (i, 0)),
            pl.BlockSpec((H_, H_), lambda i: (0, 0)),
            pl.BlockSpec((H_, H_), lambda i: (0, 0)),
        ],
        out_specs=[
            pl.BlockSpec((BM, HH), lambda i: (i, 0)),
            pl.BlockSpec((BM, HH), lambda i: (i, 0)),
            pl.BlockSpec((BM, H_), lambda i: (i, 0)),
        ],
        out_shape=[
            jax.ShapeDtypeStruct((n, HH), jnp.float32),
            jax.ShapeDtypeStruct((n, HH), jnp.float32),
            jax.ShapeDtypeStruct((n, H_), jnp.float32),
        ],
    )(x, W, Waux)


def _mid_body(na, nb, dt, b_ref, w_ref, wa_ref, ha_ref, hb_ref, aux_ref):
    d0 = jnp.sum(dt[...], axis=1, keepdims=True) + 1e-16
    h = jnp.concatenate([na[...], nb[...]], axis=1) / d0 + b_ref[...]
    h = jnp.where(h > 0, h, jnp.exp(h) - 1.0)
    h1 = jnp.dot(h, w_ref[...], preferred_element_type=jnp.float32)
    ha_ref[...] = h1[:, :HH]
    hb_ref[...] = h1[:, HH:]
    aux_ref[...] = jnp.dot(h, wa_ref[...], preferred_element_type=jnp.float32)


def _mid(num, dent, b, W, Waux):
    return pl.pallas_call(
        _mid_body,
        grid=(N_ // BM,),
        in_specs=[
            pl.BlockSpec((BM, HH), lambda i: (i, 0)),
            pl.BlockSpec((BM, HH), lambda i: (i, 0)),
            pl.BlockSpec((BM, NS), lambda i: (i, 0)),
            pl.BlockSpec((1, H_), lambda i: (0, 0)),
            pl.BlockSpec((H_, H_), lambda i: (0, 0)),
            pl.BlockSpec((H_, H_), lambda i: (0, 0)),
        ],
        out_specs=[
            pl.BlockSpec((BM, HH), lambda i: (i, 0)),
            pl.BlockSpec((BM, HH), lambda i: (i, 0)),
            pl.BlockSpec((BM, H_), lambda i: (i, 0)),
        ],
        out_shape=[
            jax.ShapeDtypeStruct((N_, HH), jnp.float32),
            jax.ShapeDtypeStruct((N_, HH), jnp.float32),
            jax.ShapeDtypeStruct((N_, H_), jnp.float32),
        ],
    )(num[0], num[1], dent, b, W, Waux)


def _head_body(na, nb, dt, b_ref, m_ref, w1_ref, b1_ref, w2_ref, wex_ref,
               ex_ref, b2_ref, pred_ref, xl_ref, sums, counts):
    i = pl.program_id(0)

    @pl.when(i == 0)
    def _():
        sums[...] = jnp.zeros_like(sums)
        counts[...] = jnp.zeros_like(counts)

    d0 = jnp.sum(dt[...], axis=1, keepdims=True) + 1e-16
    h = jnp.concatenate([na[...], nb[...]], axis=1) / d0 + b_ref[...]
    m = m_ref[...].reshape(m_ref.shape[1], m_ref.shape[2])
    sums[...] += jnp.dot(m, h, preferred_element_type=jnp.float32)
    counts[...] += jnp.broadcast_to(jnp.sum(m, axis=1, keepdims=True), counts.shape)

    @pl.when(i == pl.num_programs(0) - 1)
    def _():
        pooled = sums[...] / jnp.maximum(counts[...], 1.0)
        xl = jnp.dot(pooled, w1_ref[...], preferred_element_type=jnp.float32)
        xl = jnp.maximum(xl + b1_ref[...], 0.0)
        xl_ref[...] = xl
        logits = (jnp.dot(xl, w2_ref[...], preferred_element_type=jnp.float32)
                  + jnp.dot(ex_ref[...], wex_ref[...], preferred_element_type=jnp.float32)
                  + b2_ref[...])
        ci = lax.broadcasted_iota(jnp.int32, logits.shape, 1)
        logits = jnp.where(ci < 5, logits, -1e30)
        mx = jnp.max(logits, axis=1, keepdims=True)
        p = jnp.exp(logits - mx)
        pred_ref[...] = p / jnp.sum(p, axis=1, keepdims=True)


def _head(num, dent, b, msk, w1, b1, w2, wex, exin, b2):
    g = msk.shape[1]
    return pl.pallas_call(
        _head_body,
        grid=(N_ // BM,),
        in_specs=[
            pl.BlockSpec((BM, HH), lambda i: (i, 0)),
            pl.BlockSpec((BM, HH), lambda i: (i, 0)),
            pl.BlockSpec((BM, NS), lambda i: (i, 0)),
            pl.BlockSpec((1, H_), lambda i: (0, 0)),
            pl.BlockSpec((1, g, BM), lambda i: (i, 0, 0)),
            pl.BlockSpec((H_, H_), lambda i: (0, 0)),
            pl.BlockSpec((1, H_), lambda i: (0, 0)),
            pl.BlockSpec((H_, H_), lambda i: (0, 0)),
            pl.BlockSpec((H_, H_), lambda i: (0, 0)),
            pl.BlockSpec((g, H_), lambda i: (0, 0)),
            pl.BlockSpec((1, H_), lambda i: (0, 0)),
        ],
        out_specs=[
            pl.BlockSpec((g, H_), lambda i: (0, 0)),
            pl.BlockSpec((g, H_), lambda i: (0, 0)),
        ],
        out_shape=[jax.ShapeDtypeStruct((g, H_), jnp.float32)] * 2,
        scratch_shapes=[
            pltpu.VMEM((g, H_), jnp.float32),
            pltpu.VMEM((g, H_), jnp.float32),
        ],
    )(num[0], num[1], dent, b, msk, w1, b1, w2, wex, exin, b2)


def _sc_edge_body(ei_hbm, ha_hbm, hb_hbm, as_hbm, ad_hbm, zr_hbm, num_hbm, den_hbm,
                  as_v, ad_v, src2, dst2, rows_a, rows_b, zbuf, den_v, acc,
                  sga, sgb, ssa, ssb):
    c = lax.axis_index("c")
    s = lax.axis_index("s")

    pltpu.sync_copy(as_hbm, as_v)
    pltpu.sync_copy(ad_hbm, ad_v)
    pltpu.sync_copy(zr_hbm, zbuf)
    pltpu.sync_copy(ei_hbm.at[0, s], src2)
    pltpu.sync_copy(ei_hbm.at[1, s], dst2)

    def zden(i, cr):
        den_v[pl.ds(i * L, L)] = jnp.zeros((L,), jnp.float32)
        return cr
    lax.fori_loop(0, N_ // L, zden, 0)

    for j in range(RPT // ZR):
        pltpu.sync_copy(zbuf, acc.at[pl.ds(s * RPT + j * ZR, ZR), :])
    plsc.subcore_barrier()

    def issue_gather(i, rows, sem):
        @pl.when(c == 0)
        def _():
            pltpu.async_copy(ha_hbm.at[src2.at[i]], rows, sem)

        @pl.when(c == 1)
        def _():
            pltpu.async_copy(hb_hbm.at[src2.at[i]], rows, sem)

    def drain(rows, sem):
        pltpu.make_async_copy(ha_hbm.at[pl.ds(0, K), :], rows, sem).wait()

    def compute(i, rows):
        for g in range(K // L):
            si = src2[i, pl.ds(g * L, L)]
            di = dst2[i, pl.ds(g * L, L)]
            e = plsc.load_gather(as_v, [si]) + plsc.load_gather(ad_v, [di])
            e = jnp.where(e >= 0, e, e * 0.2)
            wvec = jnp.exp(e)
            plsc.addupdate_scatter(den_v, [di], wvec)
            for lane in range(L):
                r = g * L + lane
                wr = wvec[lane]
                for cc in range(HH // L):
                    rows[r, pl.ds(cc * L, L)] = rows[r, pl.ds(cc * L, L)] * wr

    issue_gather(0, rows_a, sga)
    issue_gather(1, rows_b, sgb)

    def pair(p, carry):
        i0 = 2 * p
        drain(rows_a, sga)
        compute(i0, rows_a)
        pltpu.async_copy(rows_a, acc.at[dst2.at[i0]], ssa, add=True)
        drain(rows_b, sgb)
        drain(rows_a, ssa)

        @pl.when(p < NCHUNK // 2 - 1)
        def _():
            issue_gather(i0 + 2, rows_a, sga)

        compute(i0 + 1, rows_b)
        pltpu.async_copy(rows_b, acc.at[dst2.at[i0 + 1]], ssb, add=True)
        drain(rows_b, ssb)

        @pl.when(p < NCHUNK // 2 - 1)
        def _():
            issue_gather(i0 + 3, rows_b, sgb)

        return carry

    lax.fori_loop(0, NCHUNK // 2, pair, 0)
    plsc.subcore_barrier()

    pltpu.sync_copy(acc.at[pl.ds(s * RPT, RPT), :],
                    num_hbm.at[c, pl.ds(s * RPT, RPT), :])

    @pl.when(c == 0)
    def _():
        pltpu.sync_copy(den_v, den_hbm.at[s])


@functools.cache
def _sc_edge():
    return pl.kernel(
        _sc_edge_body,
        out_type=[
            jax.ShapeDtypeStruct((NC, N_, HH), jnp.float32),
            jax.ShapeDtypeStruct((NS, N_), jnp.float32),
        ],
        mesh=plsc.VectorSubcoreMesh(core_axis_name="c", subcore_axis_name="s",
                                    num_cores=NC, num_subcores=NS),
        compiler_params=pltpu.CompilerParams(use_tc_tiling_on_sc=False,
                                             needs_layout_passes=False),
        scratch_types=[
            pltpu.VMEM((N_,), jnp.float32),
            pltpu.VMEM((N_,), jnp.float32),
            pltpu.VMEM((NCHUNK, K), jnp.int32),
            pltpu.VMEM((NCHUNK, K), jnp.int32),
            pltpu.VMEM((K, HH), jnp.float32),
            pltpu.VMEM((K, HH), jnp.float32),
            pltpu.VMEM((ZR, HH), jnp.float32),
            pltpu.VMEM((N_,), jnp.float32),
            pltpu.VMEM_SHARED((N_, HH), jnp.float32),
            pltpu.SemaphoreType.DMA,
            pltpu.SemaphoreType.DMA,
            pltpu.SemaphoreType.DMA,
            pltpu.SemaphoreType.DMA,
        ],
    )


def _gat_layer_sc(edge_index, ha, hb, a_s, a_d, zr):
    ei4 = edge_index.reshape(2, NS, NCHUNK, K)
    num, den = _sc_edge()(ei4, ha, hb, a_s, a_d, zr)
    return num, den.T


def kernel(x, edge_index, batch, sex, cag, W0, a_src0, a_dst0, b0,
           W1, a_src1, a_dst1, b1, lin1_W, lin1_b, lin2_W, lin2_b):
    n = x.shape[0]
    g = sex.shape[0]
    f32 = jnp.float32

    wz = jnp.zeros((H_, H_ - 2), f32)
    waux0 = jnp.concatenate([(W0 @ a_src0)[:, None], (W0 @ a_dst0)[:, None], wz], axis=1)
    waux1 = jnp.concatenate([(W1 @ a_src1)[:, None], (W1 @ a_dst1)[:, None], wz], axis=1)
    zr = jnp.zeros((ZR, HH), f32)

    h0a, h0b, aux0 = _mm2(x, W0, waux0)
    num0, dent0 = _gat_layer_sc(edge_index, h0a, h0b, aux0[:, 0], aux0[:, 1], zr)

    h1a, h1b, aux1 = _mid(num0, dent0, b0.reshape(1, H_), W1, waux1)
    num1, dent1 = _gat_layer_sc(edge_index, h1a, h1b, aux1[:, 0], aux1[:, 1], zr)

    msk = (jnp.arange(g, dtype=jnp.int32)[None, :, None]
           == batch.reshape(n // BM, 1, BM)).astype(f32)
    exin = jnp.concatenate([sex[:, None], cag[:, None], jnp.zeros((g, H_ - 2), f32)],
                           axis=1)
    lin2m = jnp.pad(lin2_W[:H_], ((0, 0), (0, H_ - 5)))
    wex = jnp.pad(lin2_W[H_:H_ + 2], ((0, H_ - 2), (0, H_ - 5)))
    lin2b_pad = jnp.pad(lin2_b, (0, H_ - 5))[None, :]
    pred_pad, x_lin1 = _head(num1, dent1, b1.reshape(1, H_), msk,
                             lin1_W, lin1_b.reshape(1, H_), lin2m, wex, exin,
                             lin2b_pad)
    prediction = pred_pad[:, :5]
    return (prediction, x_lin1)
